# baseline (speedup 1.0000x reference)
import jax
import jax.numpy as jnp
from jax import lax
from jax.experimental import pallas as pl

NUM_CLASSES = 1000
BATCH = 16384
COLS = 2048


def _onehot_block(x_ref, out_ref):
    x = x_ref[...]
    rows = lax.broadcasted_iota(jnp.int32, (NUM_CLASSES, COLS), 0)
    out_ref[...] = jnp.where(x == rows, 1.0, 0.0).astype(jnp.float32)


def kernel(x):
    x = x.astype(jnp.int32).reshape(1, BATCH)
    grid = BATCH // COLS
    oh_t = pl.pallas_call(
        _onehot_block,
        grid=(grid,),
        in_specs=[pl.BlockSpec((1, COLS), lambda i: (0, i))],
        out_specs=pl.BlockSpec((NUM_CLASSES, COLS), lambda i: (0, i)),
        out_shape=jax.ShapeDtypeStruct((NUM_CLASSES, BATCH), jnp.float32),
    )(x)
    return oh_t.T
